# baseline (speedup 1.0000x reference)
import functools

import jax
import jax.numpy as jnp
from jax import lax
from jax.experimental import pallas as pl
from jax.experimental.pallas import tpu as pltpu
from jax.experimental.pallas import tpu_sc as plsc

M_ROWS = 1048576
D = 64
B = 16384
D_IN = 128

NS = 16
ITEMS = B // NS
J = ITEMS // 128


def _matmul_body(x_ref, w_ref, o_ref):
    o_ref[...] = jnp.dot(x_ref[...], w_ref[...],
                         preferred_element_type=jnp.float32)


def _embed(x, w):
    blk = 512
    return pl.pallas_call(
        _matmul_body,
        grid=(B // blk,),
        in_specs=[
            pl.BlockSpec((blk, D_IN), lambda i: (i, 0)),
            pl.BlockSpec((D_IN, D), lambda i: (0, 0)),
        ],
        out_specs=pl.BlockSpec((blk, D), lambda i: (i, 0)),
        out_shape=jax.ShapeDtypeStruct((B, D), jnp.float32),
    )(x, w)


def _copy_row(src2d, j, dst1):
    for c in range(128 // 16):
        sl = pl.ds(c * 16, 16)
        dst1[sl] = src2d[j, sl]


def _sc_body(idx2d, zeros, emb, sarr, tag, acc,
             idxb, gb, embb, sb, zb, idx1, gb1, posw, gbw):
    wid = lax.axis_index("s")
    base = wid * ITEMS
    iota = lax.iota(jnp.int32, 16)
    zero16 = iota * 0

    pltpu.sync_copy(idx2d.at[pl.ds(wid * J, J)], idxb)
    pltpu.sync_copy(zeros, zb)

    for j in range(J):
        _copy_row(idxb, j, idx1)
        for c in range(128 // 16):
            plsc.store_scatter(posw, [iota + c * 16, zero16],
                               iota + (base + j * 128 + c * 16))
        pltpu.sync_copy(posw, tag.at[idx1])
        pltpu.sync_copy(zb, acc.at[pl.ds(base + j * 128, 128)])
    plsc.subcore_barrier()

    for j in range(J):
        _copy_row(idxb, j, idx1)
        pltpu.sync_copy(tag.at[idx1], gbw)
        for c in range(128 // 16):
            sl = pl.ds(c * 16, 16)
            gb[j, sl] = plsc.load_gather(gbw, [iota + c * 16, zero16])

    for j in range(J):
        _copy_row(gb, j, gb1)
        pltpu.sync_copy(emb.at[pl.ds(base + j * 128, 128)], embb)
        pltpu.sync_copy(embb, acc.at[gb1], add=True)
    plsc.subcore_barrier()

    for j in range(J):
        _copy_row(gb, j, gb1)
        pltpu.sync_copy(acc.at[gb1], sb)
        pltpu.sync_copy(sb, sarr.at[pl.ds(base + j * 128, 128)])


@functools.partial(
    pl.kernel,
    out_type=(
        jax.ShapeDtypeStruct((B, D), jnp.float32),
        jax.ShapeDtypeStruct((M_ROWS, 16), jnp.int32),
    ),
    mesh=plsc.VectorSubcoreMesh(core_axis_name="c", subcore_axis_name="s",
                                num_cores=1),
    compiler_params=pltpu.CompilerParams(use_tc_tiling_on_sc=False,
                                         needs_layout_passes=False),
    scratch_types=(
        pltpu.VMEM_SHARED((B, D), jnp.float32),
        pltpu.VMEM((J, 128), jnp.int32),
        pltpu.VMEM((J, 128), jnp.int32),
        pltpu.VMEM((128, D), jnp.float32),
        pltpu.VMEM((128, D), jnp.float32),
        pltpu.VMEM((128, D), jnp.float32),
        pltpu.VMEM((128,), jnp.int32),
        pltpu.VMEM((128,), jnp.int32),
        pltpu.VMEM((128, 16), jnp.int32),
        pltpu.VMEM((128, 16), jnp.int32),
    ),
)
def _sc_groupsums(idx2d, zeros, emb, sarr, tag, *scratch):
    _sc_body(idx2d, zeros, emb, sarr, tag, *scratch)


def _combine_body(mr_ref, s_ref, e_ref, o_ref):
    read = mr_ref[...] + s_ref[...]
    o_ref[...] = read * (e_ref[...] + 1.0)


def _combine(memrows, s, emb):
    blk = 2048
    spec = pl.BlockSpec((blk, D), lambda i: (i, 0))
    return pl.pallas_call(
        _combine_body,
        grid=(B // blk,),
        in_specs=[spec, spec, spec],
        out_specs=spec,
        out_shape=jax.ShapeDtypeStruct((B, D), jnp.float32),
    )(memrows, s, emb)


def kernel(mem, idx, x, W_emb):
    emb = _embed(x, W_emb)
    idx2d = idx.reshape(NS * J, 128)
    zeros = jnp.zeros((128, D), jnp.float32)
    s, _ = _sc_groupsums(idx2d, zeros, emb)
    memrows = jnp.take(mem, idx, axis=0)
    return _combine(memrows, s, emb)
